# baseline (speedup 1.0000x reference)
import functools

import jax
import jax.numpy as jnp
from jax import lax
from jax.experimental import pallas as pl
from jax.experimental.pallas import tpu as pltpu
from jax.experimental.pallas import tpu_sc as plsc

LANES = 16
IDX_CHUNK = 128


def kernel(center, context, target_table, context_table):
    B = center.shape[0]
    D = target_table.shape[1]
    info = plsc.get_sparse_core_info()
    num_workers = info.num_cores * info.num_subcores
    b_per_w = B // num_workers
    nch = b_per_w // IDX_CHUNK
    nq = D // LANES

    center_r = center.astype(jnp.int32).reshape(num_workers, nch, IDX_CHUNK)
    context_r = context.astype(jnp.int32).reshape(num_workers, nch, IDX_CHUNK)

    mesh = plsc.VectorSubcoreMesh(core_axis_name="c", subcore_axis_name="s")

    @functools.partial(
        pl.kernel,
        mesh=mesh,
        out_type=jax.ShapeDtypeStruct((B,), jnp.float32),
        compiler_params=pltpu.CompilerParams(
            needs_layout_passes=False, use_tc_tiling_on_sc=False),
        scratch_types=[
            pltpu.VMEM((nch, IDX_CHUNK), jnp.int32),
            pltpu.VMEM((nch, IDX_CHUNK), jnp.int32),
            pltpu.VMEM((b_per_w, D), jnp.float32),
            pltpu.VMEM((b_per_w, D), jnp.float32),
            pltpu.VMEM((b_per_w,), jnp.float32),
            pltpu.SemaphoreType.DMA,
            pltpu.SemaphoreType.DMA,
        ],
    )
    def sc_kernel(center_hbm, context_hbm, ttab_hbm, ctab_hbm, out_hbm,
                  cidx, xidx, arows, crows, outv, sem_a, sem_c):
        wid = lax.axis_index("s") * info.num_cores + lax.axis_index("c")
        base = wid * b_per_w
        pltpu.sync_copy(center_hbm.at[wid], cidx)
        pltpu.sync_copy(context_hbm.at[wid], xidx)
        copies = []
        for j in range(nch):
            copies.append(pltpu.async_copy(
                ttab_hbm.at[cidx.at[j]],
                arows.at[pl.ds(j * IDX_CHUNK, IDX_CHUNK)], sem_a))
            copies.append(pltpu.async_copy(
                ctab_hbm.at[xidx.at[j]],
                crows.at[pl.ds(j * IDX_CHUNK, IDX_CHUNK)], sem_c))
        for cp in copies:
            cp.wait()

        lane = lax.iota(jnp.int32, LANES)

        def group_body(g, carry):
            outvec = jnp.zeros((LANES,), jnp.float32)
            for r in range(LANES):
                b = g * LANES + r
                acc = arows[b, pl.ds(0, LANES)] * crows[b, pl.ds(0, LANES)]
                for q in range(1, nq):
                    acc = acc + (arows[b, pl.ds(q * LANES, LANES)]
                                 * crows[b, pl.ds(q * LANES, LANES)])
                tot = jnp.broadcast_to(jnp.sum(acc), (LANES,))
                outvec = jnp.where(lane == r, tot, outvec)
            outv[pl.ds(g * LANES, LANES)] = 1.0 / (1.0 + jnp.exp(-outvec))
            return carry

        lax.fori_loop(0, b_per_w // LANES, group_body, 0)
        pltpu.sync_copy(outv, out_hbm.at[pl.ds(base, b_per_w)])

    return sc_kernel(center_r, context_r, target_table, context_table)
